# baseline (speedup 1.0000x reference)
import functools

import jax
import jax.numpy as jnp
from jax.experimental import pallas as pl

EMBED_DIM = 32
N_EMBED = 1024
BM = 1024


def _vq_kernel(x_ref, embed_ref, q_ref, codes_ref, idx_ref):
    x = x_ref[...]
    embed = embed_ref[...]
    x2 = jnp.sum(x * x, axis=1, keepdims=True)
    e2 = jnp.sum(embed * embed, axis=0, keepdims=True)
    xe = jnp.dot(x, embed, preferred_element_type=jnp.float32)
    d = (x2 - 2.0 * xe) + e2
    lanes = jax.lax.broadcasted_iota(jnp.int32, (BM, N_EMBED), 1)
    idx = jnp.argmin(d, axis=1).astype(jnp.int32)
    enc = (lanes == idx[:, None]).astype(jnp.float32)
    q = jax.lax.dot_general(
        enc, embed,
        dimension_numbers=(((1,), (1,)), ((), ())),
        preferred_element_type=jnp.float32)
    q_ref[...] = x + (q - x)
    codes_ref[:, :EMBED_DIM] = x
    codes_ref[:, EMBED_DIM:] = q
    idx_ref[0, 0, :] = idx


@jax.jit
def kernel(inputs, embed):
    lead_shape = inputs.shape[:-1]
    flat = inputs.reshape(-1, EMBED_DIM)
    n = flat.shape[0]
    nblk = n // BM
    q, codes, idx3 = pl.pallas_call(
        _vq_kernel,
        grid=(nblk,),
        in_specs=[
            pl.BlockSpec((BM, EMBED_DIM), lambda i: (i, 0)),
            pl.BlockSpec((EMBED_DIM, N_EMBED), lambda i: (0, 0)),
        ],
        out_specs=[
            pl.BlockSpec((BM, EMBED_DIM), lambda i: (i, 0)),
            pl.BlockSpec((BM, 2 * EMBED_DIM), lambda i: (i, 0)),
            pl.BlockSpec((1, 1, BM), lambda i: (i, 0, 0)),
        ],
        out_shape=[
            jax.ShapeDtypeStruct((n, EMBED_DIM), jnp.float32),
            jax.ShapeDtypeStruct((n, 2 * EMBED_DIM), jnp.float32),
            jax.ShapeDtypeStruct((nblk, 1, BM), jnp.int32),
        ],
    )(flat, embed)
    quantized_st = q.reshape(*lead_shape, EMBED_DIM)
    codes_out = codes.reshape(*lead_shape, 2 * EMBED_DIM)
    indices = idx3.reshape(lead_shape)
    return (quantized_st, codes_out, indices)


# speedup vs baseline: 1.0913x; 1.0913x over previous
import functools

import jax
import jax.numpy as jnp
from jax.experimental import pallas as pl

EMBED_DIM = 32
N_EMBED = 1024
BM = 4096


def _vq_kernel(x_ref, embed_ref, q_ref, codes_ref, idx_ref):
    x = x_ref[...]
    embed = embed_ref[...]
    x2 = jnp.sum(x * x, axis=1, keepdims=True)
    e2 = jnp.sum(embed * embed, axis=0, keepdims=True)
    xe = jnp.dot(x, embed, preferred_element_type=jnp.float32)
    d = (x2 - 2.0 * xe) + e2
    lanes = jax.lax.broadcasted_iota(jnp.int32, (BM, N_EMBED), 1)
    idx = jnp.argmin(d, axis=1).astype(jnp.int32)
    enc = (lanes == idx[:, None]).astype(jnp.float32)
    q = jax.lax.dot_general(
        enc, embed,
        dimension_numbers=(((1,), (1,)), ((), ())),
        preferred_element_type=jnp.float32)
    q_ref[...] = x + (q - x)
    codes_ref[:, :EMBED_DIM] = x
    codes_ref[:, EMBED_DIM:] = q
    idx_ref[0, 0, :] = idx


@jax.jit
def kernel(inputs, embed):
    lead_shape = inputs.shape[:-1]
    flat = inputs.reshape(-1, EMBED_DIM)
    n = flat.shape[0]
    nblk = n // BM
    q, codes, idx3 = pl.pallas_call(
        _vq_kernel,
        grid=(nblk,),
        in_specs=[
            pl.BlockSpec((BM, EMBED_DIM), lambda i: (i, 0)),
            pl.BlockSpec((EMBED_DIM, N_EMBED), lambda i: (0, 0)),
        ],
        out_specs=[
            pl.BlockSpec((BM, EMBED_DIM), lambda i: (i, 0)),
            pl.BlockSpec((BM, 2 * EMBED_DIM), lambda i: (i, 0)),
            pl.BlockSpec((1, 1, BM), lambda i: (i, 0, 0)),
        ],
        out_shape=[
            jax.ShapeDtypeStruct((n, EMBED_DIM), jnp.float32),
            jax.ShapeDtypeStruct((n, 2 * EMBED_DIM), jnp.float32),
            jax.ShapeDtypeStruct((nblk, 1, BM), jnp.int32),
        ],
    )(flat, embed)
    quantized_st = q.reshape(*lead_shape, EMBED_DIM)
    codes_out = codes.reshape(*lead_shape, 2 * EMBED_DIM)
    indices = idx3.reshape(lead_shape)
    return (quantized_st, codes_out, indices)


# speedup vs baseline: 1.4366x; 1.3165x over previous
import jax
import jax.numpy as jnp
from jax.experimental import pallas as pl

EMBED_DIM = 32
N_EMBED = 1024
BM = 8192
CHUNK = 2048


def _vq_kernel(x_ref, embed_ref, qt_ref, codest_ref, idx_ref):
    embed = embed_ref[...]
    e2 = jnp.sum(embed * embed, axis=0, keepdims=True)
    lanes = jax.lax.broadcasted_iota(jnp.int32, (CHUNK, N_EMBED), 1)
    rows = jax.lax.broadcasted_iota(jnp.int32, (EMBED_DIM, EMBED_DIM), 0)
    eye = (rows == rows.T).astype(jnp.float32)
    for c in range(BM // CHUNK):
        sl = pl.ds(c * CHUNK, CHUNK)
        x = x_ref[sl, :]
        x2 = jnp.sum(x * x, axis=1, keepdims=True)
        xe = jnp.dot(x, embed, preferred_element_type=jnp.float32)
        d = (x2 - 2.0 * xe) + e2
        idx = jnp.argmin(d, axis=1).astype(jnp.int32)
        xt = jax.lax.dot_general(
            eye, x, dimension_numbers=(((1,), (1,)), ((), ())),
            preferred_element_type=jnp.float32)
        enc = (lanes == idx[:, None]).astype(jnp.float32)
        qt = jax.lax.dot_general(
            embed, enc, dimension_numbers=(((1,), (1,)), ((), ())),
            preferred_element_type=jnp.float32)
        qt_ref[:, sl] = xt + (qt - xt)
        codest_ref[:EMBED_DIM, sl] = xt
        codest_ref[EMBED_DIM:, sl] = qt
        idx_ref[0, 0, sl] = idx


@jax.jit
def kernel(inputs, embed):
    lead_shape = inputs.shape[:-1]
    flat = inputs.reshape(-1, EMBED_DIM)
    n = flat.shape[0]
    nblk = n // BM
    qt, codest, idx3 = pl.pallas_call(
        _vq_kernel,
        grid=(nblk,),
        in_specs=[
            pl.BlockSpec((BM, EMBED_DIM), lambda i: (i, 0)),
            pl.BlockSpec((EMBED_DIM, N_EMBED), lambda i: (0, 0)),
        ],
        out_specs=[
            pl.BlockSpec((EMBED_DIM, BM), lambda i: (0, i)),
            pl.BlockSpec((2 * EMBED_DIM, BM), lambda i: (0, i)),
            pl.BlockSpec((1, 1, BM), lambda i: (i, 0, 0)),
        ],
        out_shape=[
            jax.ShapeDtypeStruct((EMBED_DIM, n), jnp.float32),
            jax.ShapeDtypeStruct((2 * EMBED_DIM, n), jnp.float32),
            jax.ShapeDtypeStruct((nblk, 1, BM), jnp.int32),
        ],
    )(flat, embed)
    quantized_st = qt.T.reshape(*lead_shape, EMBED_DIM)
    codes_out = codest.T.reshape(*lead_shape, 2 * EMBED_DIM)
    indices = idx3.reshape(lead_shape)
    return (quantized_st, codes_out, indices)
